# Initial kernel scaffold; baseline (speedup 1.0000x reference)
import jax
import jax.numpy as jnp
from jax.experimental import pallas as pl


def kernel(x, edges, edge_weights, generators_nodes, uW1, ub1, uW2, ub2, uW3, ub3, rW1, rb1, rW2, rb2, rW3, rb3):
    raise NotImplementedError("write your pallas kernel here")


# speedup vs baseline: 1.0022x; 1.0022x over previous
import functools

import jax
import jax.numpy as jnp
from jax import lax
from jax.experimental import pallas as pl
from jax.experimental.pallas import tpu as pltpu

N = 50000
E = 1600000
F = 16
DE = 4
S = 32
G = 1024
ITERS = 3

MLP_R = 2000


def _mlp_body(rep_ref, ssum_ref, scnt_ref, smax_ref, smin_ref,
              w1_ref, b1_ref, w2_ref, b2_ref, w3_ref, b3_ref, out_ref):
    rep = rep_ref[...]
    cnt = jnp.maximum(scnt_ref[...], 1.0)
    am = ssum_ref[...] / cnt
    h = jnp.concatenate([rep, am, smax_ref[...], smin_ref[...]], axis=1)
    h = jnp.tanh(h @ w1_ref[...] + b1_ref[...])
    h = jnp.tanh(h @ w2_ref[...] + b2_ref[...])
    h = jnp.tanh(h @ w3_ref[...] + b3_ref[...])
    nrm = lax.rsqrt(jnp.maximum(jnp.sum(h * h, axis=-1, keepdims=True), 1e-12))
    out_ref[...] = h * nrm


def _mlp_update(rep, ssum, scnt, smax, smin, w1, b1, w2, b2, w3, b3):
    n = rep.shape[0]
    grid = n // MLP_R
    row = lambda i: (i, 0)
    full = lambda i: (0, 0)
    return pl.pallas_call(
        _mlp_body,
        grid=(grid,),
        in_specs=[
            pl.BlockSpec((MLP_R, S), row),
            pl.BlockSpec((MLP_R, S), row),
            pl.BlockSpec((MLP_R, 1), row),
            pl.BlockSpec((MLP_R, S), row),
            pl.BlockSpec((MLP_R, S), row),
            pl.BlockSpec(w1.shape, full),
            pl.BlockSpec(b1.shape, full),
            pl.BlockSpec(w2.shape, full),
            pl.BlockSpec(b2.shape, full),
            pl.BlockSpec(w3.shape, full),
            pl.BlockSpec(b3.shape, full),
        ],
        out_specs=pl.BlockSpec((MLP_R, S), row),
        out_shape=jax.ShapeDtypeStruct((n, S), jnp.float32),
    )(rep, ssum, scnt, smax, smin, w1, b1, w2, b2, w3, b3)


def _assemble_body(x_ref, wsum_ref, wcnt_ref, wmax_ref, wmin_ref, out_ref):
    cnt = jnp.maximum(wcnt_ref[...], 1.0)
    wm = wsum_ref[...] / cnt
    out_ref[...] = jnp.concatenate(
        [x_ref[...], wm, wmax_ref[...], wmin_ref[...], wsum_ref[...]], axis=1)


def _assemble(x, wsum, wcnt, wmax, wmin):
    n = x.shape[0]
    grid = n // MLP_R
    row = lambda i: (i, 0)
    return pl.pallas_call(
        _assemble_body,
        grid=(grid,),
        in_specs=[
            pl.BlockSpec((MLP_R, F), row),
            pl.BlockSpec((MLP_R, DE), row),
            pl.BlockSpec((MLP_R, 1), row),
            pl.BlockSpec((MLP_R, DE), row),
            pl.BlockSpec((MLP_R, DE), row),
        ],
        out_specs=pl.BlockSpec((MLP_R, S), row),
        out_shape=jax.ShapeDtypeStruct((n, S), jnp.float32),
    )(x, wsum, wcnt, wmax, wmin)


def _readout_body(gen_ref, w1_ref, b1_ref, w2_ref, b2_ref, w3_ref, b3_ref, out_ref):
    g = gen_ref[...]
    g = jnp.where(jnp.isnan(g), jnp.zeros_like(g), g)
    r = jnp.tanh(g @ w1_ref[...] + b1_ref[...])
    r = jnp.tanh(r @ w2_ref[...] + b2_ref[...])
    out_ref[...] = r @ w3_ref[...] + b3_ref[...]


def _readout(gen, w1, b1, w2, b2, w3, b3):
    return pl.pallas_call(
        _readout_body,
        out_shape=jax.ShapeDtypeStruct((G, 1), jnp.float32),
    )(gen, w1, b1, w2, b2, w3, b3)


def _seg_stats(data, ids, n):
    s = jax.ops.segment_sum(data, ids, num_segments=n)
    c = jax.ops.segment_sum(jnp.ones((data.shape[0], 1), data.dtype), ids,
                            num_segments=n)
    mx = jax.ops.segment_max(data, ids, num_segments=n)
    mn = jax.ops.segment_min(data, ids, num_segments=n)
    return s, c, mx, mn


def kernel(x, edges, edge_weights, generators_nodes,
           uW1, ub1, uW2, ub2, uW3, ub3, rW1, rb1, rW2, rb2, rW3, rb3):
    src = edges[0]
    nb = edges[1]
    ub1r, ub2r, ub3r = ub1[None, :], ub2[None, :], ub3[None, :]
    rb1r, rb2r, rb3r = rb1[None, :], rb2[None, :], rb3[None, :]

    ws, wc, wmax, wmin = _seg_stats(edge_weights, src, N)
    rep = _assemble(x, ws, wc, wmax, wmin)

    for _ in range(ITERS):
        msgs = jnp.take(rep, nb, axis=0)
        s, c, mx, mn = _seg_stats(msgs, src, N)
        rep = _mlp_update(rep, s, c, mx, mn, uW1, ub1r, uW2, ub2r, uW3, ub3r)

    gen = jnp.take(rep, generators_nodes, axis=0)
    vals = _readout(gen, rW1, rb1r, rW2, rb2r, rW3, rb3r)
    return jnp.reshape(vals, (-1,))


# speedup vs baseline: 2.4882x; 2.4827x over previous
import jax
import jax.numpy as jnp
from jax import lax
from jax.experimental import pallas as pl
from jax.experimental.pallas import tpu as pltpu
from jax.experimental.pallas import tpu_sc as plsc

N = 50000
E = 1600000
F = 16
DE = 4
S = 32
G = 1024
ITERS = 3

NT = 32
RB = 512
NB = (N + RB - 1) // RB
NBW = 4
NPAD = NB * RB
TILE_E = E // NT
CHUNK_P = 2000
NCHUNK_P = TILE_E // CHUNK_P
BUF = 256
BSTR = BUF + 16
CAP = TILE_E + BSTR
CHUNK_G = 64
CHUNK_G_LOG = 6
CNTW = 128

MLP_R = 2000

_MESH = plsc.VectorSubcoreMesh(core_axis_name="c", subcore_axis_name="s")


def _wid():
    return lax.axis_index("s") * 2 + lax.axis_index("c")


def _al8(x):
    return pl.multiple_of(x, 8)


def _partition_body(src_hbm, nb_hbm, pnb_hbm, peid_hbm, cnt_hbm,
                    srcv, nbv, bufn, bufe, cntv, pos_s, sem):
    t = _wid()
    iota = jnp.arange(16, dtype=jnp.int32)
    tbase = t * (NB * CAP)

    def zb(b, carry):
        pos_s[b] = 0
        return carry
    lax.fori_loop(0, NB, zb, 0)

    def fire(ci, slot):
        base = t * TILE_E + ci * CHUNK_P
        pltpu.async_copy(src_hbm.at[pl.ds(_al8(base), CHUNK_P)],
                         srcv.at[pl.ds(_al8(slot * CHUNK_P), CHUNK_P)], sem)
        pltpu.async_copy(nb_hbm.at[pl.ds(_al8(base), CHUNK_P)],
                         nbv.at[pl.ds(_al8(slot * CHUNK_P), CHUNK_P)], sem)

    def drain(ci, slot):
        base = t * TILE_E + ci * CHUNK_P
        pltpu.make_async_copy(
            src_hbm.at[pl.ds(_al8(base), CHUNK_P)],
            srcv.at[pl.ds(_al8(slot * CHUNK_P), CHUNK_P)], sem).wait()
        pltpu.make_async_copy(
            nb_hbm.at[pl.ds(_al8(base), CHUNK_P)],
            nbv.at[pl.ds(_al8(slot * CHUNK_P), CHUNK_P)], sem).wait()

    fire(0, 0)

    def chunk_body(ci, carry):
        slot = jnp.bitwise_and(ci, 1)
        drain(ci, slot)

        @pl.when(ci + 1 < NCHUNK_P)
        def _prefetch():
            fire(ci + 1, 1 - slot)

        ebase = t * TILE_E + ci * CHUNK_P

        def group(g, c2):
            off = g * 16
            sv = srcv[pl.ds(slot * CHUNK_P + off, 16)]
            nv = nbv[pl.ds(slot * CHUNK_P + off, 16)]
            for j in range(16):
                s = sv[j]
                b = lax.shift_right_logical(s, 9)
                sl = s - lax.shift_left(b, 9)
                pk1 = jnp.bitwise_or(nv[j], lax.shift_left(sl, 16))
                pk2 = jnp.bitwise_or(ebase + off + j, lax.shift_left(sl, 21))
                p = pos_s[b]
                lo = jnp.bitwise_and(p, BUF - 1)
                bufn[pl.ds(b * BSTR + lo, 16)] = jnp.broadcast_to(pk1, (16,))
                bufe[pl.ds(b * BSTR + lo, 16)] = jnp.broadcast_to(pk2, (16,))
                pos_s[b] = p + 1

                @pl.when(lo == BUF - 1)
                def _flush(b=b, p=p):
                    w = p - (BUF - 1)
                    pltpu.sync_copy(
                        bufn.at[pl.ds(b * BSTR, BUF)],
                        pnb_hbm.at[pl.ds(_al8(tbase + b * CAP + w), BUF)])
                    pltpu.sync_copy(
                        bufe.at[pl.ds(b * BSTR, BUF)],
                        peid_hbm.at[pl.ds(_al8(tbase + b * CAP + w), BUF)])
            return c2

        lax.fori_loop(0, CHUNK_P // 16, group, 0)
        return carry

    lax.fori_loop(0, NCHUNK_P, chunk_body, 0)

    def fin(b, carry):
        p = pos_s[b]
        w = p - jnp.bitwise_and(p, BUF - 1)
        pltpu.sync_copy(bufn.at[pl.ds(b * BSTR, BSTR)],
                        pnb_hbm.at[pl.ds(_al8(tbase + b * CAP + w), BSTR)])
        pltpu.sync_copy(bufe.at[pl.ds(b * BSTR, BSTR)],
                        peid_hbm.at[pl.ds(_al8(tbase + b * CAP + w), BSTR)])
        return carry
    lax.fori_loop(0, NB, fin, 0)

    zero16 = jnp.zeros((16,), jnp.int32)
    for gi in range(CNTW // 16):
        v = zero16
        for j in range(16):
            idx = gi * 16 + j
            if idx < NB:
                v = jnp.where(iota == j, jnp.broadcast_to(pos_s[idx], (16,)), v)
        cntv[pl.ds(gi * 16, 16)] = v
    pltpu.sync_copy(cntv, cnt_hbm.at[pl.ds(_al8(t * CNTW), CNTW)])


def _partition(src, nb):
    buf_t = pltpu.VMEM((NB * BSTR,), jnp.int32)
    return pl.kernel(
        _partition_body,
        out_type=(
            jax.ShapeDtypeStruct((NT * NB * CAP,), jnp.int32),
            jax.ShapeDtypeStruct((NT * NB * CAP,), jnp.int32),
            jax.ShapeDtypeStruct((NT * CNTW,), jnp.int32),
        ),
        mesh=_MESH,
        scratch_types=(
            pltpu.VMEM((2 * CHUNK_P,), jnp.int32),
            pltpu.VMEM((2 * CHUNK_P,), jnp.int32),
            buf_t, buf_t,
            pltpu.VMEM((CNTW,), jnp.int32),
            pltpu.SMEM((NB,), jnp.int32),
            pltpu.SemaphoreType.DMA,
        ),
    )(src, nb)


def _seg_body(idxmask, slshift, rowshift, submask, submul, halves,
              packed_hbm, cnt_hbm, data_hbm,
              sum_hbm, cntf_hbm, max_hbm, min_hbm,
              accs, accx, accn, accf, cnts_v, pk_v, idx_v, slv, rows_v,
              accc, sem):
    wid = _wid()
    pltpu.sync_copy(cnt_hbm, cnts_v.at[pl.ds(0, NT * CNTW)])
    iota = jnp.arange(16, dtype=jnp.int32)
    zero16 = jnp.zeros((16,), jnp.float32)
    ninf16 = jnp.full((16,), -jnp.inf, jnp.float32)
    pinf16 = jnp.full((16,), jnp.inf, jnp.float32)

    aw = 16 * halves

    def do_bucket(b):
        def init_row(i, carry):
            for h in range(halves):
                accs[pl.ds(i * aw + h * 16, 16)] = zero16
                accx[pl.ds(i * aw + h * 16, 16)] = ninf16
                accn[pl.ds(i * aw + h * 16, 16)] = pinf16
            return carry
        lax.fori_loop(0, RB + 1, init_row, 0)

        def init_c(i, carry):
            accc[i] = 0
            return carry
        lax.fori_loop(0, RB + 1, init_c, 0)

        def build(slot, ebase, cnt):
            def g_body(g, c3):
                off = g * 16
                lane = iota + (ebase + off)
                pk = pk_v[pl.ds(slot * CHUNK_G + off, 16)]
                m = lane < cnt
                idxs = jnp.where(m, jnp.bitwise_and(pk, idxmask), 0)
                sls = jnp.where(
                    m, lax.shift_right_logical(pk, slshift), RB)
                rows = lax.shift_right_logical(idxs, rowshift)
                sub = jnp.bitwise_and(idxs, submask) * submul
                idx_v[pl.ds(slot * CHUNK_G + off, 16)] = rows
                slv[pl.ds(slot * CHUNK_G + off, 16)] = jnp.bitwise_or(
                    sls, lax.shift_left(sub, 12))
                return c3
            lax.fori_loop(0, CHUNK_G // 16, g_body, 0)

        def accum(slot):
            def e_body(g, c3):
                off = g * 16
                ovec = slv[pl.ds(slot * CHUNK_G + off, 16)]
                for j in range(16):
                    o = ovec[j]
                    sub = lax.shift_right_logical(o, 12)
                    sl = jnp.bitwise_and(o, 0xFFF)
                    ao = sl * aw
                    r2d = slot * CHUNK_G + off + j
                    r0 = rows_v[r2d, pl.ds(sub, 16)]
                    accs[pl.ds(ao, 16)] = accs[pl.ds(ao, 16)] + r0
                    accx[pl.ds(ao, 16)] = jnp.maximum(accx[pl.ds(ao, 16)], r0)
                    accn[pl.ds(ao, 16)] = jnp.minimum(accn[pl.ds(ao, 16)], r0)
                    if halves == 2:
                        r1 = rows_v[r2d, pl.ds(sub + 16, 16)]
                        accs[pl.ds(ao + 16, 16)] = accs[pl.ds(ao + 16, 16)] + r1
                        accx[pl.ds(ao + 16, 16)] = jnp.maximum(
                            accx[pl.ds(ao + 16, 16)], r1)
                        accn[pl.ds(ao + 16, 16)] = jnp.minimum(
                            accn[pl.ds(ao + 16, 16)], r1)
                    accc[sl] = accc[sl] + 1
                return c3
            lax.fori_loop(0, CHUNK_G // 16, e_body, 0)

        def t2_body(t2, carry):
            cnt = cnts_v[pl.ds(t2 * CNTW + b, 16)][0]
            nch = lax.shift_right_logical(cnt + (CHUNK_G - 1), CHUNK_G_LOG)
            sbase = t2 * (NB * CAP) + b * CAP

            @pl.when(nch > 0)
            def _run():
                pltpu.sync_copy(
                    packed_hbm.at[pl.ds(_al8(sbase), CHUNK_G)],
                    pk_v.at[pl.ds(0, CHUNK_G)])

                def ch_body(ch, c2):
                    slot = jnp.bitwise_and(ch, 1)
                    ebase = ch * CHUNK_G
                    build(slot, ebase, cnt)
                    h = pltpu.async_copy(
                        data_hbm.at[idx_v.at[pl.ds(_al8(slot * CHUNK_G),
                                                   CHUNK_G)]],
                        rows_v.at[pl.ds(_al8(slot * CHUNK_G), CHUNK_G)], sem)

                    @pl.when(ch + 1 < nch)
                    def _pref():
                        pltpu.sync_copy(
                            packed_hbm.at[pl.ds(
                                _al8(sbase + (ch + 1) * CHUNK_G), CHUNK_G)],
                            pk_v.at[pl.ds(_al8((1 - slot) * CHUNK_G),
                                          CHUNK_G)])

                    @pl.when(ch > 0)
                    def _acc_prev():
                        accum(1 - slot)

                    h.wait()
                    return c2
                lax.fori_loop(0, nch, ch_body, 0)
                accum(jnp.bitwise_and(nch - 1, 1))
            return carry
        lax.fori_loop(0, NT, t2_body, 0)

        def cw(gi, carry):
            v = zero16
            for j in range(16):
                v = jnp.where(iota == j,
                              jnp.broadcast_to(
                                  accc[gi * 16 + j].astype(jnp.float32),
                                  (16,)), v)
            accf[pl.ds(gi * 16, 16)] = v
            return carry
        lax.fori_loop(0, RB // 16, cw, 0)

        base = b * RB
        pltpu.sync_copy(accs.at[pl.ds(0, RB * aw)],
                        sum_hbm.at[pl.ds(_al8(base * aw), RB * aw)])
        pltpu.sync_copy(accx.at[pl.ds(0, RB * aw)],
                        max_hbm.at[pl.ds(_al8(base * aw), RB * aw)])
        pltpu.sync_copy(accn.at[pl.ds(0, RB * aw)],
                        min_hbm.at[pl.ds(_al8(base * aw), RB * aw)])
        pltpu.sync_copy(accf.at[pl.ds(0, RB)],
                        cntf_hbm.at[pl.ds(_al8(base), RB)])

    do_bucket(wid)
    for w in range(1, NBW):
        b2 = wid + w * NT
        if NB > w * NT:
            @pl.when(b2 < NB)
            def _wave(b2=b2):
                do_bucket(b2)


def _seg_reduce(packed, cnts, data, idxmask, slshift, rowshift, submask,
                submul, halves):
    aw = 16 * halves

    def body(*refs):
        _seg_body(idxmask, slshift, rowshift, submask, submul, halves, *refs)
    s_, c_, x_, n_ = pl.kernel(
        body,
        out_type=(
            jax.ShapeDtypeStruct((NPAD * aw,), jnp.float32),
            jax.ShapeDtypeStruct((NPAD,), jnp.float32),
            jax.ShapeDtypeStruct((NPAD * aw,), jnp.float32),
            jax.ShapeDtypeStruct((NPAD * aw,), jnp.float32),
        ),
        mesh=_MESH,
        scratch_types=(
            pltpu.VMEM(((RB + 1) * aw,), jnp.float32),
            pltpu.VMEM(((RB + 1) * aw,), jnp.float32),
            pltpu.VMEM(((RB + 1) * aw,), jnp.float32),
            pltpu.VMEM((RB,), jnp.float32),
            pltpu.VMEM((NT * CNTW + 16,), jnp.int32),
            pltpu.VMEM((2 * CHUNK_G,), jnp.int32),
            pltpu.VMEM((2 * CHUNK_G,), jnp.int32),
            pltpu.VMEM((2 * CHUNK_G,), jnp.int32),
            pltpu.VMEM((2 * CHUNK_G, 128), jnp.float32),
            pltpu.SMEM((RB + 1,), jnp.int32),
            pltpu.SemaphoreType.DMA,
        ),
    )(packed, cnts, data)
    return (jnp.reshape(s_, (NPAD, aw)), c_,
            jnp.reshape(x_, (NPAD, aw)), jnp.reshape(n_, (NPAD, aw)))


def _gather_body(tbl_hbm, idx_hbm, out_hbm, idxv, rowv, outv, sem):
    wid = _wid()
    npt = G // NT
    base = wid * npt
    pltpu.sync_copy(idx_hbm.at[pl.ds(_al8(base), npt)],
                    idxv.at[pl.ds(0, npt)])

    def bld(g, carry):
        off = g * 16
        gv = idxv[pl.ds(off, 16)]
        idxv[pl.ds(npt + off, 16)] = lax.shift_right_logical(gv, 2)
        return carry
    lax.fori_loop(0, npt // 16, bld, 0)
    pltpu.async_copy(tbl_hbm.at[idxv.at[pl.ds(_al8(npt), npt)]],
                     rowv, sem).wait()
    for g in range(npt // 16):
        gv = idxv[pl.ds(g * 16, 16)]
        for j in range(16):
            sub = jnp.bitwise_and(gv[j], 3) * 32
            outv[g * 16 + j, pl.ds(0, 16)] = rowv[g * 16 + j, pl.ds(sub, 16)]
            outv[g * 16 + j, pl.ds(16, 16)] = rowv[g * 16 + j,
                                                   pl.ds(sub + 16, 16)]
    pltpu.sync_copy(outv, out_hbm.at[pl.ds(_al8(base), npt)])


def _gather_gen(tbl, idx):
    npt = G // NT
    return pl.kernel(
        _gather_body,
        out_type=jax.ShapeDtypeStruct((G, S), jnp.float32),
        mesh=_MESH,
        scratch_types=(
            pltpu.VMEM((2 * npt,), jnp.int32),
            pltpu.VMEM((npt, 128), jnp.float32),
            pltpu.VMEM((npt, S), jnp.float32),
            pltpu.SemaphoreType.DMA,
        ),
    )(tbl, idx)


def _mlp_body(rep_ref, ssum_ref, scnt_ref, smax_ref, smin_ref,
              w1_ref, b1_ref, w2_ref, b2_ref, w3_ref, b3_ref, out_ref):
    rep = rep_ref[...]
    cnt = jnp.maximum(scnt_ref[...], 1.0)
    am = ssum_ref[...] / cnt
    h = jnp.concatenate([rep, am, smax_ref[...], smin_ref[...]], axis=1)
    h = jnp.tanh(h @ w1_ref[...] + b1_ref[...])
    h = jnp.tanh(h @ w2_ref[...] + b2_ref[...])
    h = jnp.tanh(h @ w3_ref[...] + b3_ref[...])
    nrm = lax.rsqrt(jnp.maximum(jnp.sum(h * h, axis=-1, keepdims=True), 1e-12))
    out_ref[...] = h * nrm


def _mlp_update(rep, ssum, scnt, smax, smin, w1, b1, w2, b2, w3, b3):
    n = rep.shape[0]
    grid = n // MLP_R
    row = lambda i: (i, 0)
    full = lambda i: (0, 0)
    return pl.pallas_call(
        _mlp_body,
        grid=(grid,),
        in_specs=[
            pl.BlockSpec((MLP_R, S), row),
            pl.BlockSpec((MLP_R, S), row),
            pl.BlockSpec((MLP_R, 1), row),
            pl.BlockSpec((MLP_R, S), row),
            pl.BlockSpec((MLP_R, S), row),
            pl.BlockSpec(w1.shape, full),
            pl.BlockSpec(b1.shape, full),
            pl.BlockSpec(w2.shape, full),
            pl.BlockSpec(b2.shape, full),
            pl.BlockSpec(w3.shape, full),
            pl.BlockSpec(b3.shape, full),
        ],
        out_specs=pl.BlockSpec((MLP_R, S), row),
        out_shape=jax.ShapeDtypeStruct((n, S), jnp.float32),
    )(rep, ssum, scnt, smax, smin, w1, b1, w2, b2, w3, b3)


def _assemble_body(x_ref, wsum_ref, wcnt_ref, wmax_ref, wmin_ref, out_ref):
    cnt = jnp.maximum(wcnt_ref[...], 1.0)
    wm = wsum_ref[...] / cnt
    out_ref[...] = jnp.concatenate(
        [x_ref[...], wm, wmax_ref[...], wmin_ref[...], wsum_ref[...]], axis=1)


def _assemble(x, wsum, wcnt, wmax, wmin):
    n = x.shape[0]
    grid = n // MLP_R
    row = lambda i: (i, 0)
    return pl.pallas_call(
        _assemble_body,
        grid=(grid,),
        in_specs=[
            pl.BlockSpec((MLP_R, F), row),
            pl.BlockSpec((MLP_R, DE), row),
            pl.BlockSpec((MLP_R, 1), row),
            pl.BlockSpec((MLP_R, DE), row),
            pl.BlockSpec((MLP_R, DE), row),
        ],
        out_specs=pl.BlockSpec((MLP_R, S), row),
        out_shape=jax.ShapeDtypeStruct((n, S), jnp.float32),
    )(x, wsum, wcnt, wmax, wmin)


def _readout_body(gen_ref, w1_ref, b1_ref, w2_ref, b2_ref, w3_ref, b3_ref,
                  out_ref):
    g = gen_ref[...]
    g = jnp.where(jnp.isnan(g), jnp.zeros_like(g), g)
    r = jnp.tanh(g @ w1_ref[...] + b1_ref[...])
    r = jnp.tanh(r @ w2_ref[...] + b2_ref[...])
    out_ref[...] = r @ w3_ref[...] + b3_ref[...]


def _readout(gen, w1, b1, w2, b2, w3, b3):
    return pl.pallas_call(
        _readout_body,
        out_shape=jax.ShapeDtypeStruct((G, 1), jnp.float32),
    )(gen, w1, b1, w2, b2, w3, b3)


def kernel(x, edges, edge_weights, generators_nodes,
           uW1, ub1, uW2, ub2, uW3, ub3, rW1, rb1, rW2, rb2, rW3, rb3):
    src = edges[0]
    nb = edges[1]
    ub1r, ub2r, ub3r = ub1[None, :], ub2[None, :], ub3[None, :]
    rb1r, rb2r, rb3r = rb1[None, :], rb2[None, :], rb3[None, :]

    pnb, peid, cnts = _partition(src, nb)

    ew16 = jnp.pad(edge_weights, ((0, 0), (0, 16 - DE)))
    ew128 = jnp.reshape(ew16, (E // 8, 128))
    ps, pc, px, pn = _seg_reduce(peid, cnts, ew128, idxmask=0x1FFFFF,
                                 slshift=21, rowshift=3, submask=7,
                                 submul=16, halves=1)
    rep = _assemble(x, ps[:N, :DE], pc[:N, None], px[:N, :DE], pn[:N, :DE])

    for _ in range(ITERS):
        rep128 = jnp.reshape(rep, (N // 4, 128))
        s, c, mx, mn = _seg_reduce(pnb, cnts, rep128, idxmask=0xFFFF,
                                   slshift=16, rowshift=2, submask=3,
                                   submul=32, halves=2)
        rep = _mlp_update(rep, s[:N], c[:N, None], mx[:N], mn[:N],
                          uW1, ub1r, uW2, ub2r, uW3, ub3r)

    gen = _gather_gen(jnp.reshape(rep, (N // 4, 128)), generators_nodes)
    vals = _readout(gen, rW1, rb1r, rW2, rb2r, rW3, rb3r)
    return jnp.reshape(vals, (-1,))
